# baseline (speedup 1.0000x reference)
import jax
import jax.numpy as jnp
from jax.experimental import pallas as pl
from jax.experimental.pallas import tpu as pltpu

_LAMBDA = 0.1
_SMOOTH = 1e-07


def _body(B, C, S, num_h):
    def body(sup_ref, cs_ref, s2s_ref, tgt_ref, w_ref,
             loss_ref, fin_ref, oh_ref,
             a_interc, a_sumc, a_cntc, a_intsup, a_sums):
        b = pl.program_id(0)
        h = pl.program_id(1)

        @pl.when(jnp.logical_and(b == 0, h == 0))
        def _init():
            for c in range(C):
                a_interc[c] = 0.0
                a_sumc[c] = 0.0
                a_cntc[c] = 0.0
                a_intsup[c] = 0.0
            for s in range(S):
                a_sums[s] = 0.0

        t = tgt_ref[0]
        ones_col = jnp.ones((t.shape[1], 1), jnp.float32)

        def rsum(a):
            col = jax.lax.dot_general(
                a, ones_col, (((1,), (0,)), ((), ())),
                preferred_element_type=jnp.float32)
            return jnp.sum(col)

        def class_body(c, carry):
            oh = t == c
            ohf = oh.astype(jnp.float32)
            oh_ref[0, c] = ohf
            x = cs_ref[0, c]
            sidx = s2s_ref[c]
            g = sup_ref[0, sidx]
            fin_ref[0, c] = x * g
            a_interc[c] += rsum(x * ohf)
            a_sumc[c] += rsum(x)
            a_cntc[c] += rsum(ohf)
            a_intsup[c] += rsum(g * ohf)
            return carry

        jax.lax.fori_loop(0, C, class_body, 0)
        for s in range(S):
            a_sums[s] += rsum(sup_ref[0, s])

        @pl.when(jnp.logical_and(b == B - 1, h == num_h - 1))
        def _finish():
            sl = 0.0
            for s in range(S):
                cnt_s = 0.0
                int_s = 0.0
                for c in range(C):
                    pred = s2s_ref[c] == s
                    cnt_s += jnp.where(pred, a_cntc[c], 0.0)
                    int_s += jnp.where(pred, a_intsup[c], 0.0)
                sl += 1.0 - (2.0 * int_s + _SMOOTH) / (
                    a_sums[s] + cnt_s + _SMOOTH)
            cl = 0.0
            wsum = 0.0
            for c in range(C):
                pc = 1.0 - (2.0 * a_interc[c] + _SMOOTH) / (
                    a_sumc[c] + a_cntc[c] + _SMOOTH)
                cl += pc * w_ref[c]
                wsum += w_ref[c]
            loss_ref[0, 0] = _LAMBDA * sl / S + cl / wsum

    return body


def kernel(superclass_scores, class_score, super2sub, target, weights):
    B, C, H, W = class_score.shape
    S = superclass_scores.shape[1]
    bh = 256
    num_h = H // bh

    cids = jnp.arange(C, dtype=jnp.int32)
    member = jnp.any(super2sub.astype(jnp.int32)[None, :, :] == cids[:, None, None],
                     axis=2)
    sub2super = jnp.sum(member.astype(jnp.int32)
                        * jnp.arange(S, dtype=jnp.int32)[None, :], axis=1)

    grid = (B, num_h)
    out_shapes = (
        jax.ShapeDtypeStruct((1, 1), jnp.float32),
        jax.ShapeDtypeStruct((B, C, H, W), jnp.float32),
        jax.ShapeDtypeStruct((B, C, H, W), jnp.float32),
    )
    loss2d, fin, oh = pl.pallas_call(
        _body(B, C, S, num_h),
        grid=grid,
        in_specs=[
            pl.BlockSpec((1, S, bh, W), lambda b, h: (b, 0, h, 0)),
            pl.BlockSpec((1, C, bh, W), lambda b, h: (b, 0, h, 0)),
            pl.BlockSpec(memory_space=pltpu.SMEM),
            pl.BlockSpec((1, bh, W), lambda b, h: (b, h, 0)),
            pl.BlockSpec(memory_space=pltpu.SMEM),
        ],
        out_specs=(
            pl.BlockSpec(memory_space=pltpu.SMEM),
            pl.BlockSpec((1, C, bh, W), lambda b, h: (b, 0, h, 0)),
            pl.BlockSpec((1, C, bh, W), lambda b, h: (b, 0, h, 0)),
        ),
        scratch_shapes=[
            pltpu.SMEM((C,), jnp.float32),
            pltpu.SMEM((C,), jnp.float32),
            pltpu.SMEM((C,), jnp.float32),
            pltpu.SMEM((C,), jnp.float32),
            pltpu.SMEM((S,), jnp.float32),
        ],
        out_shape=out_shapes,
    )(superclass_scores, class_score, sub2super, target, weights)
    return (loss2d.reshape(()), fin, oh)


# speedup vs baseline: 1.2821x; 1.2821x over previous
import jax
import jax.numpy as jnp
from jax.experimental import pallas as pl
from jax.experimental.pallas import tpu as pltpu

_LAMBDA = 0.1
_SMOOTH = 1e-07


def _body(B, C, S, num_h):
    def body(sup_ref, cs_ref, s2s_ref, tgt_ref, w_ref,
             loss_ref, fin_ref, oh_ref,
             a_interc, a_sumc, a_cntc, a_intsup, a_sums):
        b = pl.program_id(0)
        h = pl.program_id(1)

        @pl.when(jnp.logical_and(b == 0, h == 0))
        def _init():
            for c in range(C):
                a_interc[c] = 0.0
                a_sumc[c] = 0.0
                a_cntc[c] = 0.0
                a_intsup[c] = 0.0
            for s in range(S):
                a_sums[s] = 0.0

        t = tgt_ref[0]

        def class_body(c, carry):
            oh = t == c
            ohf = oh.astype(jnp.float32)
            oh_ref[0, c] = ohf
            x = cs_ref[0, c]
            sidx = s2s_ref[c]
            g = sup_ref[0, sidx]
            fin_ref[0, c] = x * g
            a_interc[c] += jnp.sum(x * ohf)
            a_sumc[c] += jnp.sum(x)
            a_cntc[c] += jnp.sum(ohf)
            a_intsup[c] += jnp.sum(g * ohf)
            return carry

        jax.lax.fori_loop(0, C, class_body, 0, unroll=4)
        for s in range(S):
            a_sums[s] += jnp.sum(sup_ref[0, s])

        @pl.when(jnp.logical_and(b == B - 1, h == num_h - 1))
        def _finish():
            sl = 0.0
            for s in range(S):
                cnt_s = 0.0
                int_s = 0.0
                for c in range(C):
                    pred = s2s_ref[c] == s
                    cnt_s += jnp.where(pred, a_cntc[c], 0.0)
                    int_s += jnp.where(pred, a_intsup[c], 0.0)
                sl += 1.0 - (2.0 * int_s + _SMOOTH) / (
                    a_sums[s] + cnt_s + _SMOOTH)
            cl = 0.0
            wsum = 0.0
            for c in range(C):
                pc = 1.0 - (2.0 * a_interc[c] + _SMOOTH) / (
                    a_sumc[c] + a_cntc[c] + _SMOOTH)
                cl += pc * w_ref[c]
                wsum += w_ref[c]
            loss_ref[0, 0] = _LAMBDA * sl / S + cl / wsum

    return body


def kernel(superclass_scores, class_score, super2sub, target, weights):
    B, C, H, W = class_score.shape
    S = superclass_scores.shape[1]
    bh = 256
    num_h = H // bh

    cids = jnp.arange(C, dtype=jnp.int32)
    member = jnp.any(super2sub.astype(jnp.int32)[None, :, :] == cids[:, None, None],
                     axis=2)
    sub2super = jnp.sum(member.astype(jnp.int32)
                        * jnp.arange(S, dtype=jnp.int32)[None, :], axis=1)

    grid = (B, num_h)
    out_shapes = (
        jax.ShapeDtypeStruct((1, 1), jnp.float32),
        jax.ShapeDtypeStruct((B, C, H, W), jnp.float32),
        jax.ShapeDtypeStruct((B, C, H, W), jnp.float32),
    )
    loss2d, fin, oh = pl.pallas_call(
        _body(B, C, S, num_h),
        grid=grid,
        in_specs=[
            pl.BlockSpec((1, S, bh, W), lambda b, h: (b, 0, h, 0)),
            pl.BlockSpec((1, C, bh, W), lambda b, h: (b, 0, h, 0)),
            pl.BlockSpec(memory_space=pltpu.SMEM),
            pl.BlockSpec((1, bh, W), lambda b, h: (b, h, 0)),
            pl.BlockSpec(memory_space=pltpu.SMEM),
        ],
        out_specs=(
            pl.BlockSpec(memory_space=pltpu.SMEM),
            pl.BlockSpec((1, C, bh, W), lambda b, h: (b, 0, h, 0)),
            pl.BlockSpec((1, C, bh, W), lambda b, h: (b, 0, h, 0)),
        ),
        scratch_shapes=[
            pltpu.SMEM((C,), jnp.float32),
            pltpu.SMEM((C,), jnp.float32),
            pltpu.SMEM((C,), jnp.float32),
            pltpu.SMEM((C,), jnp.float32),
            pltpu.SMEM((S,), jnp.float32),
        ],
        out_shape=out_shapes,
    )(superclass_scores, class_score, sub2super, target, weights)
    return (loss2d.reshape(()), fin, oh)


# speedup vs baseline: 1.2831x; 1.0008x over previous
import jax
import jax.numpy as jnp
from jax.experimental import pallas as pl
from jax.experimental.pallas import tpu as pltpu

_LAMBDA = 0.1
_SMOOTH = 1e-07


def _body(B, C, S, num_h):
    def body(sup_ref, cs_ref, s2s_ref, tgt_ref, w_ref,
             loss_ref, fin_ref, oh_ref,
             a_interc, a_sumc, a_cntc, a_intsup, a_sums):
        b = pl.program_id(0)
        h = pl.program_id(1)

        @pl.when(jnp.logical_and(b == 0, h == 0))
        def _init():
            for c in range(C):
                a_interc[c] = 0.0
                a_sumc[c] = 0.0
                a_cntc[c] = 0.0
                a_intsup[c] = 0.0
            for s in range(S):
                a_sums[s] = 0.0

        t = tgt_ref[0]

        def class_body(c, carry):
            oh = t == c
            ohf = oh.astype(jnp.float32)
            oh_ref[0, c] = ohf
            x = cs_ref[0, c]
            sidx = s2s_ref[c]
            g = sup_ref[0, sidx]
            fin_ref[0, c] = x * g
            a_interc[c] += jnp.sum(x * ohf)
            a_sumc[c] += jnp.sum(x)
            a_cntc[c] += jnp.sum(ohf)
            a_intsup[c] += jnp.sum(g * ohf)
            return carry

        jax.lax.fori_loop(0, C, class_body, 0, unroll=8)
        for s in range(S):
            a_sums[s] += jnp.sum(sup_ref[0, s])

        @pl.when(jnp.logical_and(b == B - 1, h == num_h - 1))
        def _finish():
            sl = 0.0
            for s in range(S):
                cnt_s = 0.0
                int_s = 0.0
                for c in range(C):
                    pred = s2s_ref[c] == s
                    cnt_s += jnp.where(pred, a_cntc[c], 0.0)
                    int_s += jnp.where(pred, a_intsup[c], 0.0)
                sl += 1.0 - (2.0 * int_s + _SMOOTH) / (
                    a_sums[s] + cnt_s + _SMOOTH)
            cl = 0.0
            wsum = 0.0
            for c in range(C):
                pc = 1.0 - (2.0 * a_interc[c] + _SMOOTH) / (
                    a_sumc[c] + a_cntc[c] + _SMOOTH)
                cl += pc * w_ref[c]
                wsum += w_ref[c]
            loss_ref[0, 0] = _LAMBDA * sl / S + cl / wsum

    return body


def kernel(superclass_scores, class_score, super2sub, target, weights):
    B, C, H, W = class_score.shape
    S = superclass_scores.shape[1]
    bh = 256
    num_h = H // bh

    cids = jnp.arange(C, dtype=jnp.int32)
    member = jnp.any(super2sub.astype(jnp.int32)[None, :, :] == cids[:, None, None],
                     axis=2)
    sub2super = jnp.sum(member.astype(jnp.int32)
                        * jnp.arange(S, dtype=jnp.int32)[None, :], axis=1)

    grid = (B, num_h)
    out_shapes = (
        jax.ShapeDtypeStruct((1, 1), jnp.float32),
        jax.ShapeDtypeStruct((B, C, H, W), jnp.float32),
        jax.ShapeDtypeStruct((B, C, H, W), jnp.float32),
    )
    loss2d, fin, oh = pl.pallas_call(
        _body(B, C, S, num_h),
        grid=grid,
        in_specs=[
            pl.BlockSpec((1, S, bh, W), lambda b, h: (b, 0, h, 0)),
            pl.BlockSpec((1, C, bh, W), lambda b, h: (b, 0, h, 0)),
            pl.BlockSpec(memory_space=pltpu.SMEM),
            pl.BlockSpec((1, bh, W), lambda b, h: (b, h, 0)),
            pl.BlockSpec(memory_space=pltpu.SMEM),
        ],
        out_specs=(
            pl.BlockSpec(memory_space=pltpu.SMEM),
            pl.BlockSpec((1, C, bh, W), lambda b, h: (b, 0, h, 0)),
            pl.BlockSpec((1, C, bh, W), lambda b, h: (b, 0, h, 0)),
        ),
        scratch_shapes=[
            pltpu.SMEM((C,), jnp.float32),
            pltpu.SMEM((C,), jnp.float32),
            pltpu.SMEM((C,), jnp.float32),
            pltpu.SMEM((C,), jnp.float32),
            pltpu.SMEM((S,), jnp.float32),
        ],
        out_shape=out_shapes,
    )(superclass_scores, class_score, sub2super, target, weights)
    return (loss2d.reshape(()), fin, oh)
